# baseline (speedup 1.0000x reference)
import functools

import jax
import jax.numpy as jnp
from jax.experimental import pallas as pl
from jax.experimental.pallas import tpu as pltpu

_EPS = 1e-5


def _smem():
    return pl.BlockSpec(memory_space=pltpu.MemorySpace.SMEM)


def _cparams():
    return pltpu.CompilerParams(dimension_semantics=("parallel",),
                                vmem_limit_bytes=48 * 1024 * 1024)


def _edge_masks(H, W, HW):
    lane = jax.lax.broadcasted_iota(jnp.int32, (1, HW), 1)
    col = jax.lax.rem(lane, W)
    row = lane // W
    return (col > 0), (col < W - 1), (row > 0), (row < H - 1)


def _shift_cols(x, m_wm, m_wp, HW):
    xm = jnp.where(m_wm, pltpu.roll(x, shift=1, axis=1), 0.0)
    xp = jnp.where(m_wp, pltpu.roll(x, shift=HW - 1, axis=1), 0.0)
    return xm, xp


def _row_combine(q_up, q_mid, q_dn, m_hm, m_hp, W, HW):
    out = q_mid + jnp.where(m_hm, pltpu.roll(q_up, shift=W, axis=1), 0.0)
    return out + jnp.where(m_hp, pltpu.roll(q_dn, shift=HW - W, axis=1), 0.0)


def _conv3x3_s(x, xm, xp, w9, bias, masks, W, HW):
    m_wm, m_wp, m_hm, m_hp = masks
    qs = [w9[i * 3] * xm + w9[i * 3 + 1] * x + w9[i * 3 + 2] * xp
          for i in range(3)]
    y = _row_combine(qs[0], qs[1], qs[2], m_hm, m_hp, W, HW)
    return jnp.maximum(y + bias, 0.0)


def _shift_w3(v, d):
    nb, H, W = v.shape
    z = jnp.zeros((nb, H, 1), jnp.float32)
    if d == -1:
        return jnp.concatenate([z, v[:, :, :W - 1]], axis=2)
    return jnp.concatenate([v[:, :, 1:], z], axis=2)


def _shift_h3(v, d):
    nb, H, W = v.shape
    z = jnp.zeros((nb, 1, W), jnp.float32)
    if d == -1:
        return jnp.concatenate([z, v[:, :H - 1, :]], axis=1)
    return jnp.concatenate([v[:, 1:, :], z], axis=1)


def _conv3x3_hw(x, xm, xp, w9, bias):
    qs = [w9[i * 3] * xm + w9[i * 3 + 1] * x + w9[i * 3 + 2] * xp
          for i in range(3)]
    y = qs[1] + _shift_h3(qs[0], -1) + _shift_h3(qs[2], 1)
    return jnp.maximum(y + bias, 0.0)


def _stage1_body(x_ref,
                 prew_ref, preb_ref, preg_ref, prebe_ref, prem_ref, prev_ref,
                 caw_ref, cab_ref, cag_ref, cabe_ref, cam_ref, cav_ref,
                 cbw_ref, cbb_ref, cbg_ref, cbbe_ref, cbm_ref, cbv_ref,
                 a1w_ref, a1b_ref, ag_ref, abe_ref, am_ref, av_ref,
                 achw_ref, achb_ref, acww_ref, acwb_ref,
                 t1w_ref, t1b_ref, t2w_ref, t2b_ref,
                 x2_ref, t_ref, *, H, W):
    nb, Cin = x_ref.shape[0], x_ref.shape[1]
    HW = H * W

    pscale = preg_ref[0] * jax.lax.rsqrt(prev_ref[0] + _EPS)
    acc = (prew_ref[0] * pscale) * x_ref[:, 0, :, :]
    for c in range(1, Cin):
        acc = acc + (prew_ref[c] * pscale) * x_ref[:, c, :, :]
    pbias = (preb_ref[0] - prem_ref[0]) * pscale + prebe_ref[0]
    x1 = jnp.maximum(acc + pbias, 0.0)

    cas = cag_ref[0] * jax.lax.rsqrt(cav_ref[0] + _EPS)
    caw = [caw_ref[k] * cas for k in range(9)]
    cabias = (cab_ref[0] - cam_ref[0]) * cas + cabe_ref[0]
    cbs = cbg_ref[0] * jax.lax.rsqrt(cbv_ref[0] + _EPS)
    cbw = [cbw_ref[k] * cbs for k in range(9)]
    cbbias = (cbb_ref[0] - cbm_ref[0]) * cbs + cbbe_ref[0]

    y = _conv3x3_hw(x1, _shift_w3(x1, -1), _shift_w3(x1, 1), caw, cabias)
    y = _conv3x3_hw(y, _shift_w3(y, -1), _shift_w3(y, 1), cbw, cbbias)
    x2 = x1 + y
    x2_ref[...] = x2

    xh = jnp.sum(x2, axis=2) * (1.0 / W)
    xw = jnp.sum(x2, axis=1) * (1.0 / H)
    xcat = jnp.concatenate([xh, xw], axis=1)
    lane = jax.lax.broadcasted_iota(jnp.int32, (1, H + W), 1)
    is_h = lane < H
    acc_a = None
    for m in range(8):
        asc = ag_ref[m] * jax.lax.rsqrt(av_ref[m] + _EPS)
        wm = a1w_ref[m] * asc
        bm = (a1b_ref[m] - am_ref[m]) * asc + abe_ref[m]
        z = wm * xcat + bm
        z = z * jnp.clip(z + 3.0, 0.0, 6.0) * (1.0 / 6.0)
        contrib = jnp.where(is_h, achw_ref[m], acww_ref[m]) * z
        acc_a = contrib if acc_a is None else acc_a + contrib
    bsel = jnp.where(is_h, achb_ref[0], acwb_ref[0])
    a_cat = 1.0 / (1.0 + jnp.exp(-(acc_a + bsel)))
    a_h = a_cat[:, :H]
    a_w = a_cat[:, H:]

    row = jnp.sum(x2 * a_w[:, None, :], axis=2)
    sq = jnp.sum(row * a_h, axis=1, keepdims=True) * (1.0 / HW)

    tt = jnp.maximum(sq * t1w_ref[0] + t1b_ref[0], 0.0)
    tt = 1.0 / (1.0 + jnp.exp(-(tt * t2w_ref[0] + t2b_ref[0])))
    t_ref[...] = tt.reshape(t_ref.shape)


def _stage2_body(x2_ref, t_ref, w1_ref, b1_ref, g1_ref, be1_ref, m1_ref, v1_ref,
                 w2_ref, b2_ref, g2_ref, be2_ref, m2_ref, v2_ref,
                 out_ref, *, H, W, Cout):
    nb, _, HW = x2_ref.shape
    N = t_ref.shape[0]
    masks = _edge_masks(H, W, HW)
    m_wm, m_wp, m_hm, m_hp = masks

    hi = jnp.sum(t_ref[...]) * (1.0 / N)
    xc = jnp.minimum(jnp.maximum(x2_ref[:, 0, :], 1e-6), hi)

    xm, xp = _shift_cols(xc, m_wm, m_wp, HW)
    mids = []
    for o in range(Cout):
        s1 = g1_ref[o] * jax.lax.rsqrt(v1_ref[o] + _EPS)
        w9 = [w1_ref[o * 9 + k] * s1 for k in range(9)]
        b1e = (b1_ref[o] - m1_ref[o]) * s1 + be1_ref[o]
        mids.append(_conv3x3_s(xc, xm, xp, w9, b1e, masks, W, HW))

    cms, cps = [], []
    for c in range(Cout):
        cm, cp = _shift_cols(mids[c], m_wm, m_wp, HW)
        cms.append(cm)
        cps.append(cp)
    for o in range(Cout):
        s2 = g2_ref[o] * jax.lax.rsqrt(v2_ref[o] + _EPS)
        qs = [None] * 3
        for c in range(Cout):
            base = (o * Cout + c) * 9
            for i in range(3):
                q = ((w2_ref[base + i * 3] * s2) * cms[c]
                     + (w2_ref[base + i * 3 + 1] * s2) * mids[c]
                     + (w2_ref[base + i * 3 + 2] * s2) * cps[c])
                qs[i] = q if qs[i] is None else qs[i] + q
        b2e = (b2_ref[o] - m2_ref[o]) * s2 + be2_ref[o]
        y = _row_combine(qs[0], qs[1], qs[2], m_hm, m_hp, W, HW)
        out_ref[:, o, :] = jnp.maximum(y + b2e, 0.0)


def _pick_nb(n, cap=8):
    for d in range(min(cap, n), 0, -1):
        if n % d == 0:
            return d
    return n


def kernel(x, pre_w, pre_b, pre_bn_gamma, pre_bn_beta, pre_bn_mean, pre_bn_var,
           ca_w, ca_b, ca_bn_gamma, ca_bn_beta, ca_bn_mean, ca_bn_var,
           cb_w, cb_b, cb_bn_gamma, cb_bn_beta, cb_bn_mean, cb_bn_var,
           att_c1_w, att_c1_b, att_bn_gamma, att_bn_beta, att_bn_mean,
           att_bn_var, att_ch_w, att_ch_b, att_cw_w, att_cw_b,
           th1_w, th1_b, th2_w, th2_b,
           post1_w, post1_b, post1_bn_gamma, post1_bn_beta, post1_bn_mean,
           post1_bn_var, post2_w, post2_b, post2_bn_gamma, post2_bn_beta,
           post2_bn_mean, post2_bn_var):
    N, Cin, H, W = x.shape
    HW = H * W
    Cout = int(post1_w.shape[0])
    nb1 = _pick_nb(N, cap=4)
    nb2 = _pick_nb(N, cap=4)

    f32 = jnp.float32
    s1_kern = functools.partial(_stage1_body, H=H, W=W)
    x2_3d, t = pl.pallas_call(
        s1_kern,
        out_shape=(jax.ShapeDtypeStruct((N, H, W), f32),
                   jax.ShapeDtypeStruct((N, 1, 1), f32)),
        grid=(N // nb1,),
        in_specs=[
            pl.BlockSpec((nb1, Cin, H, W), lambda n: (n, 0, 0, 0)),
        ] + [_smem()] * 32,
        out_specs=(pl.BlockSpec((nb1, H, W), lambda n: (n, 0, 0)),
                   pl.BlockSpec((nb1, 1, 1), lambda n: (n, 0, 0))),
        compiler_params=_cparams(),
    )(x,
      pre_w.reshape(-1).astype(f32), pre_b.reshape(-1).astype(f32),
      pre_bn_gamma.astype(f32), pre_bn_beta.astype(f32),
      pre_bn_mean.astype(f32), pre_bn_var.astype(f32),
      ca_w.reshape(-1).astype(f32), ca_b.reshape(-1).astype(f32),
      ca_bn_gamma.astype(f32), ca_bn_beta.astype(f32),
      ca_bn_mean.astype(f32), ca_bn_var.astype(f32),
      cb_w.reshape(-1).astype(f32), cb_b.reshape(-1).astype(f32),
      cb_bn_gamma.astype(f32), cb_bn_beta.astype(f32),
      cb_bn_mean.astype(f32), cb_bn_var.astype(f32),
      att_c1_w.reshape(-1).astype(f32), att_c1_b.reshape(-1).astype(f32),
      att_bn_gamma.astype(f32), att_bn_beta.astype(f32),
      att_bn_mean.astype(f32), att_bn_var.astype(f32),
      att_ch_w.reshape(-1).astype(f32), att_ch_b.reshape(-1).astype(f32),
      att_cw_w.reshape(-1).astype(f32), att_cw_b.reshape(-1).astype(f32),
      th1_w.reshape(-1).astype(f32), th1_b.reshape(-1).astype(f32),
      th2_w.reshape(-1).astype(f32), th2_b.reshape(-1).astype(f32))

    x2 = x2_3d.reshape(N, 1, HW)
    s2_kern = functools.partial(_stage2_body, H=H, W=W, Cout=Cout)
    out = pl.pallas_call(
        s2_kern,
        out_shape=jax.ShapeDtypeStruct((N, Cout, HW), f32),
        grid=(N // nb2,),
        in_specs=[
            pl.BlockSpec((nb2, 1, HW), lambda n: (n, 0, 0)),
            pl.BlockSpec((N, 1, 1), lambda n: (0, 0, 0)),
        ] + [_smem()] * 12,
        out_specs=pl.BlockSpec((nb2, Cout, HW), lambda n: (n, 0, 0)),
        compiler_params=_cparams(),
    )(x2, t,
      post1_w.reshape(-1).astype(f32), post1_b.reshape(-1).astype(f32),
      post1_bn_gamma.astype(f32), post1_bn_beta.astype(f32),
      post1_bn_mean.astype(f32), post1_bn_var.astype(f32),
      post2_w.reshape(-1).astype(f32), post2_b.reshape(-1).astype(f32),
      post2_bn_gamma.astype(f32), post2_bn_beta.astype(f32),
      post2_bn_mean.astype(f32), post2_bn_var.astype(f32))

    return out.reshape(N, Cout, H, W)


# speedup vs baseline: 1.0580x; 1.0580x over previous
import functools

import jax
import jax.numpy as jnp
from jax.experimental import pallas as pl
from jax.experimental.pallas import tpu as pltpu

_EPS = 1e-5


def _smem():
    return pl.BlockSpec(memory_space=pltpu.MemorySpace.SMEM)


def _cparams():
    return pltpu.CompilerParams(dimension_semantics=("parallel",),
                                vmem_limit_bytes=48 * 1024 * 1024)


def _edge_masks(H, W, HW):
    lane = jax.lax.broadcasted_iota(jnp.int32, (1, HW), 1)
    col = jax.lax.rem(lane, W)
    row = lane // W
    return (col > 0), (col < W - 1), (row > 0), (row < H - 1)


def _shift_cols(x, m_wm, m_wp, HW):
    xm = jnp.where(m_wm, pltpu.roll(x, shift=1, axis=1), 0.0)
    xp = jnp.where(m_wp, pltpu.roll(x, shift=HW - 1, axis=1), 0.0)
    return xm, xp


def _row_combine(q_up, q_mid, q_dn, m_hm, m_hp, W, HW):
    out = q_mid + jnp.where(m_hm, pltpu.roll(q_up, shift=W, axis=1), 0.0)
    return out + jnp.where(m_hp, pltpu.roll(q_dn, shift=HW - W, axis=1), 0.0)


def _conv3x3_s(x, xm, xp, w9, bias, masks, W, HW):
    m_wm, m_wp, m_hm, m_hp = masks
    qs = [w9[i * 3] * xm + w9[i * 3 + 1] * x + w9[i * 3 + 2] * xp
          for i in range(3)]
    y = _row_combine(qs[0], qs[1], qs[2], m_hm, m_hp, W, HW)
    return jnp.maximum(y + bias, 0.0)


def _shift_w3(v, d):
    nb, H, W = v.shape
    z = jnp.zeros((nb, H, 1), jnp.float32)
    if d == -1:
        return jnp.concatenate([z, v[:, :, :W - 1]], axis=2)
    return jnp.concatenate([v[:, :, 1:], z], axis=2)


def _shift_h3(v, d):
    nb, H, W = v.shape
    z = jnp.zeros((nb, 1, W), jnp.float32)
    if d == -1:
        return jnp.concatenate([z, v[:, :H - 1, :]], axis=1)
    return jnp.concatenate([v[:, 1:, :], z], axis=1)


def _conv3x3_hw(x, xm, xp, w9, bias):
    qs = [w9[i * 3] * xm + w9[i * 3 + 1] * x + w9[i * 3 + 2] * xp
          for i in range(3)]
    y = qs[1] + _shift_h3(qs[0], -1) + _shift_h3(qs[2], 1)
    return jnp.maximum(y + bias, 0.0)


def _stage1_body(x_ref,
                 prew_ref, preb_ref, preg_ref, prebe_ref, prem_ref, prev_ref,
                 caw_ref, cab_ref, cag_ref, cabe_ref, cam_ref, cav_ref,
                 cbw_ref, cbb_ref, cbg_ref, cbbe_ref, cbm_ref, cbv_ref,
                 a1w_ref, a1b_ref, ag_ref, abe_ref, am_ref, av_ref,
                 achw_ref, achb_ref, acww_ref, acwb_ref,
                 t1w_ref, t1b_ref, t2w_ref, t2b_ref,
                 x2_ref, t_ref, *, H, W):
    nb, Cin = x_ref.shape[0], x_ref.shape[1]
    HW = H * W

    pscale = preg_ref[0] * jax.lax.rsqrt(prev_ref[0] + _EPS)
    acc = (prew_ref[0] * pscale) * x_ref[:, 0, :, :]
    for c in range(1, Cin):
        acc = acc + (prew_ref[c] * pscale) * x_ref[:, c, :, :]
    pbias = (preb_ref[0] - prem_ref[0]) * pscale + prebe_ref[0]
    x1 = jnp.maximum(acc + pbias, 0.0)

    cas = cag_ref[0] * jax.lax.rsqrt(cav_ref[0] + _EPS)
    caw = [caw_ref[k] * cas for k in range(9)]
    cabias = (cab_ref[0] - cam_ref[0]) * cas + cabe_ref[0]
    cbs = cbg_ref[0] * jax.lax.rsqrt(cbv_ref[0] + _EPS)
    cbw = [cbw_ref[k] * cbs for k in range(9)]
    cbbias = (cbb_ref[0] - cbm_ref[0]) * cbs + cbbe_ref[0]

    y = _conv3x3_hw(x1, _shift_w3(x1, -1), _shift_w3(x1, 1), caw, cabias)
    y = _conv3x3_hw(y, _shift_w3(y, -1), _shift_w3(y, 1), cbw, cbbias)
    x2 = x1 + y
    x2_ref[...] = x2

    xh = jnp.sum(x2, axis=2) * (1.0 / W)
    xw = jnp.sum(x2, axis=1) * (1.0 / H)
    xcat = jnp.concatenate([xh, xw], axis=1)
    lane = jax.lax.broadcasted_iota(jnp.int32, (1, H + W), 1)
    is_h = lane < H
    acc_a = None
    for m in range(8):
        asc = ag_ref[m] * jax.lax.rsqrt(av_ref[m] + _EPS)
        wm = a1w_ref[m] * asc
        bm = (a1b_ref[m] - am_ref[m]) * asc + abe_ref[m]
        z = wm * xcat + bm
        z = z * jnp.clip(z + 3.0, 0.0, 6.0) * (1.0 / 6.0)
        contrib = jnp.where(is_h, achw_ref[m], acww_ref[m]) * z
        acc_a = contrib if acc_a is None else acc_a + contrib
    bsel = jnp.where(is_h, achb_ref[0], acwb_ref[0])
    a_cat = 1.0 / (1.0 + jnp.exp(-(acc_a + bsel)))
    a_h = a_cat[:, :H]
    a_w = a_cat[:, H:]

    row = jnp.sum(x2 * a_w[:, None, :], axis=2)
    sq = jnp.sum(row * a_h, axis=1, keepdims=True) * (1.0 / HW)

    tt = jnp.maximum(sq * t1w_ref[0] + t1b_ref[0], 0.0)
    tt = 1.0 / (1.0 + jnp.exp(-(tt * t2w_ref[0] + t2b_ref[0])))
    t_ref[...] = tt.reshape(t_ref.shape)


def _stage2_body(x2_ref, t_ref, w1_ref, b1_ref, g1_ref, be1_ref, m1_ref, v1_ref,
                 w2_ref, b2_ref, g2_ref, be2_ref, m2_ref, v2_ref,
                 out_ref, *, H, W, Cout):
    nb, _, HW = x2_ref.shape
    N = t_ref.shape[0]
    masks = _edge_masks(H, W, HW)
    m_wm, m_wp, m_hm, m_hp = masks

    hi = jnp.sum(t_ref[...]) * (1.0 / N)
    xc = jnp.minimum(jnp.maximum(x2_ref[:, 0, :], 1e-6), hi)

    xm, xp = _shift_cols(xc, m_wm, m_wp, HW)
    mids = []
    for o in range(Cout):
        s1 = g1_ref[o] * jax.lax.rsqrt(v1_ref[o] + _EPS)
        w9 = [w1_ref[o * 9 + k] * s1 for k in range(9)]
        b1e = (b1_ref[o] - m1_ref[o]) * s1 + be1_ref[o]
        mids.append(_conv3x3_s(xc, xm, xp, w9, b1e, masks, W, HW))

    cms, cps = [], []
    for c in range(Cout):
        cm, cp = _shift_cols(mids[c], m_wm, m_wp, HW)
        cms.append(cm)
        cps.append(cp)
    for o in range(Cout):
        s2 = g2_ref[o] * jax.lax.rsqrt(v2_ref[o] + _EPS)
        qs = [None] * 3
        for c in range(Cout):
            base = (o * Cout + c) * 9
            for i in range(3):
                q = ((w2_ref[base + i * 3] * s2) * cms[c]
                     + (w2_ref[base + i * 3 + 1] * s2) * mids[c]
                     + (w2_ref[base + i * 3 + 2] * s2) * cps[c])
                qs[i] = q if qs[i] is None else qs[i] + q
        b2e = (b2_ref[o] - m2_ref[o]) * s2 + be2_ref[o]
        y = _row_combine(qs[0], qs[1], qs[2], m_hm, m_hp, W, HW)
        out_ref[:, o, :] = jnp.maximum(y + b2e, 0.0)


def _pick_nb(n, cap=8):
    for d in range(min(cap, n), 0, -1):
        if n % d == 0:
            return d
    return n


def kernel(x, pre_w, pre_b, pre_bn_gamma, pre_bn_beta, pre_bn_mean, pre_bn_var,
           ca_w, ca_b, ca_bn_gamma, ca_bn_beta, ca_bn_mean, ca_bn_var,
           cb_w, cb_b, cb_bn_gamma, cb_bn_beta, cb_bn_mean, cb_bn_var,
           att_c1_w, att_c1_b, att_bn_gamma, att_bn_beta, att_bn_mean,
           att_bn_var, att_ch_w, att_ch_b, att_cw_w, att_cw_b,
           th1_w, th1_b, th2_w, th2_b,
           post1_w, post1_b, post1_bn_gamma, post1_bn_beta, post1_bn_mean,
           post1_bn_var, post2_w, post2_b, post2_bn_gamma, post2_bn_beta,
           post2_bn_mean, post2_bn_var):
    N, Cin, H, W = x.shape
    HW = H * W
    Cout = int(post1_w.shape[0])
    nb1 = _pick_nb(N, cap=8)
    nb2 = _pick_nb(N, cap=4)

    f32 = jnp.float32
    s1_kern = functools.partial(_stage1_body, H=H, W=W)
    x2_3d, t = pl.pallas_call(
        s1_kern,
        out_shape=(jax.ShapeDtypeStruct((N, H, W), f32),
                   jax.ShapeDtypeStruct((N, 1, 1), f32)),
        grid=(N // nb1,),
        in_specs=[
            pl.BlockSpec((nb1, Cin, H, W), lambda n: (n, 0, 0, 0)),
        ] + [_smem()] * 32,
        out_specs=(pl.BlockSpec((nb1, H, W), lambda n: (n, 0, 0)),
                   pl.BlockSpec((nb1, 1, 1), lambda n: (n, 0, 0))),
        compiler_params=_cparams(),
    )(x,
      pre_w.reshape(-1).astype(f32), pre_b.reshape(-1).astype(f32),
      pre_bn_gamma.astype(f32), pre_bn_beta.astype(f32),
      pre_bn_mean.astype(f32), pre_bn_var.astype(f32),
      ca_w.reshape(-1).astype(f32), ca_b.reshape(-1).astype(f32),
      ca_bn_gamma.astype(f32), ca_bn_beta.astype(f32),
      ca_bn_mean.astype(f32), ca_bn_var.astype(f32),
      cb_w.reshape(-1).astype(f32), cb_b.reshape(-1).astype(f32),
      cb_bn_gamma.astype(f32), cb_bn_beta.astype(f32),
      cb_bn_mean.astype(f32), cb_bn_var.astype(f32),
      att_c1_w.reshape(-1).astype(f32), att_c1_b.reshape(-1).astype(f32),
      att_bn_gamma.astype(f32), att_bn_beta.astype(f32),
      att_bn_mean.astype(f32), att_bn_var.astype(f32),
      att_ch_w.reshape(-1).astype(f32), att_ch_b.reshape(-1).astype(f32),
      att_cw_w.reshape(-1).astype(f32), att_cw_b.reshape(-1).astype(f32),
      th1_w.reshape(-1).astype(f32), th1_b.reshape(-1).astype(f32),
      th2_w.reshape(-1).astype(f32), th2_b.reshape(-1).astype(f32))

    x2 = x2_3d.reshape(N, 1, HW)
    s2_kern = functools.partial(_stage2_body, H=H, W=W, Cout=Cout)
    out = pl.pallas_call(
        s2_kern,
        out_shape=jax.ShapeDtypeStruct((N, Cout, HW), f32),
        grid=(N // nb2,),
        in_specs=[
            pl.BlockSpec((nb2, 1, HW), lambda n: (n, 0, 0)),
            pl.BlockSpec((N, 1, 1), lambda n: (0, 0, 0)),
        ] + [_smem()] * 12,
        out_specs=pl.BlockSpec((nb2, Cout, HW), lambda n: (n, 0, 0)),
        compiler_params=_cparams(),
    )(x2, t,
      post1_w.reshape(-1).astype(f32), post1_b.reshape(-1).astype(f32),
      post1_bn_gamma.astype(f32), post1_bn_beta.astype(f32),
      post1_bn_mean.astype(f32), post1_bn_var.astype(f32),
      post2_w.reshape(-1).astype(f32), post2_b.reshape(-1).astype(f32),
      post2_bn_gamma.astype(f32), post2_bn_beta.astype(f32),
      post2_bn_mean.astype(f32), post2_bn_var.astype(f32))

    return out.reshape(N, Cout, H, W)


# speedup vs baseline: 1.3172x; 1.2450x over previous
import functools

import jax
import jax.numpy as jnp
from jax.experimental import pallas as pl
from jax.experimental.pallas import tpu as pltpu

_EPS = 1e-5


def _smem():
    return pl.BlockSpec(memory_space=pltpu.MemorySpace.SMEM)


def _cparams():
    return pltpu.CompilerParams(dimension_semantics=("parallel",),
                                vmem_limit_bytes=48 * 1024 * 1024)


def _edge_masks(H, W, HW):
    lane = jax.lax.broadcasted_iota(jnp.int32, (1, HW), 1)
    col = jax.lax.rem(lane, W)
    row = lane // W
    return (col > 0), (col < W - 1), (row > 0), (row < H - 1)


def _shift_cols(x, m_wm, m_wp, HW):
    xm = jnp.where(m_wm, pltpu.roll(x, shift=1, axis=1), 0.0)
    xp = jnp.where(m_wp, pltpu.roll(x, shift=HW - 1, axis=1), 0.0)
    return xm, xp


def _row_combine(q_up, q_mid, q_dn, m_hm, m_hp, W, HW):
    out = q_mid + jnp.where(m_hm, pltpu.roll(q_up, shift=W, axis=1), 0.0)
    return out + jnp.where(m_hp, pltpu.roll(q_dn, shift=HW - W, axis=1), 0.0)


def _conv3x3_s(x, xm, xp, w9, bias, masks, W, HW):
    m_wm, m_wp, m_hm, m_hp = masks
    qs = [w9[i * 3] * xm + w9[i * 3 + 1] * x + w9[i * 3 + 2] * xp
          for i in range(3)]
    y = _row_combine(qs[0], qs[1], qs[2], m_hm, m_hp, W, HW)
    return jnp.maximum(y + bias, 0.0)


def _shift_w3(v, d):
    nb, H, W = v.shape
    z = jnp.zeros((nb, H, 1), jnp.float32)
    if d == -1:
        return jnp.concatenate([z, v[:, :, :W - 1]], axis=2)
    return jnp.concatenate([v[:, :, 1:], z], axis=2)


def _shift_h3(v, d):
    nb, H, W = v.shape
    z = jnp.zeros((nb, 1, W), jnp.float32)
    if d == -1:
        return jnp.concatenate([z, v[:, :H - 1, :]], axis=1)
    return jnp.concatenate([v[:, 1:, :], z], axis=1)


def _conv3x3_hw(x, xm, xp, w9, bias):
    qs = [w9[i * 3] * xm + w9[i * 3 + 1] * x + w9[i * 3 + 2] * xp
          for i in range(3)]
    y = qs[1] + _shift_h3(qs[0], -1) + _shift_h3(qs[2], 1)
    return jnp.maximum(y + bias, 0.0)


def _stage1_body(x_ref,
                 prew_ref, preb_ref, preg_ref, prebe_ref, prem_ref, prev_ref,
                 caw_ref, cab_ref, cag_ref, cabe_ref, cam_ref, cav_ref,
                 cbw_ref, cbb_ref, cbg_ref, cbbe_ref, cbm_ref, cbv_ref,
                 a1w_ref, a1b_ref, ag_ref, abe_ref, am_ref, av_ref,
                 achw_ref, achb_ref, acww_ref, acwb_ref,
                 t1w_ref, t1b_ref, t2w_ref, t2b_ref,
                 x2_ref, t_ref, *, H, W):
    nb, Cin = x_ref.shape[0], x_ref.shape[1]
    HW = H * W

    pscale = preg_ref[0] * jax.lax.rsqrt(prev_ref[0] + _EPS)
    acc = (prew_ref[0] * pscale) * x_ref[:, 0, :, :]
    for c in range(1, Cin):
        acc = acc + (prew_ref[c] * pscale) * x_ref[:, c, :, :]
    pbias = (preb_ref[0] - prem_ref[0]) * pscale + prebe_ref[0]
    x1 = jnp.maximum(acc + pbias, 0.0)

    cas = cag_ref[0] * jax.lax.rsqrt(cav_ref[0] + _EPS)
    caw = [caw_ref[k] * cas for k in range(9)]
    cabias = (cab_ref[0] - cam_ref[0]) * cas + cabe_ref[0]
    cbs = cbg_ref[0] * jax.lax.rsqrt(cbv_ref[0] + _EPS)
    cbw = [cbw_ref[k] * cbs for k in range(9)]
    cbbias = (cbb_ref[0] - cbm_ref[0]) * cbs + cbbe_ref[0]

    y = _conv3x3_hw(x1, _shift_w3(x1, -1), _shift_w3(x1, 1), caw, cabias)
    y = _conv3x3_hw(y, _shift_w3(y, -1), _shift_w3(y, 1), cbw, cbbias)
    x2 = x1 + y
    x2_ref[...] = x2

    xh = jnp.sum(x2, axis=2) * (1.0 / W)
    xw = jnp.sum(x2, axis=1) * (1.0 / H)
    xcat = jnp.concatenate([xh, xw], axis=1)
    lane = jax.lax.broadcasted_iota(jnp.int32, (1, H + W), 1)
    is_h = lane < H
    acc_a = None
    for m in range(8):
        asc = ag_ref[m] * jax.lax.rsqrt(av_ref[m] + _EPS)
        wm = a1w_ref[m] * asc
        bm = (a1b_ref[m] - am_ref[m]) * asc + abe_ref[m]
        z = wm * xcat + bm
        z = z * jnp.clip(z + 3.0, 0.0, 6.0) * (1.0 / 6.0)
        contrib = jnp.where(is_h, achw_ref[m], acww_ref[m]) * z
        acc_a = contrib if acc_a is None else acc_a + contrib
    bsel = jnp.where(is_h, achb_ref[0], acwb_ref[0])
    a_cat = 1.0 / (1.0 + jnp.exp(-(acc_a + bsel)))
    a_h = a_cat[:, :H]
    a_w = a_cat[:, H:]

    row = jnp.sum(x2 * a_w[:, None, :], axis=2)
    sq = jnp.sum(row * a_h, axis=1, keepdims=True) * (1.0 / HW)

    tt = jnp.maximum(sq * t1w_ref[0] + t1b_ref[0], 0.0)
    tt = 1.0 / (1.0 + jnp.exp(-(tt * t2w_ref[0] + t2b_ref[0])))
    t_ref[...] = tt.reshape(t_ref.shape)


def _stage2_body(x2_ref, t_ref, w1_ref, b1_ref, g1_ref, be1_ref, m1_ref, v1_ref,
                 w2_ref, b2_ref, g2_ref, be2_ref, m2_ref, v2_ref,
                 out_ref, *, H, W, Cout):
    nb, _, HW = x2_ref.shape
    N = t_ref.shape[0]
    masks = _edge_masks(H, W, HW)
    m_wm, m_wp, m_hm, m_hp = masks

    hi = jnp.sum(t_ref[...]) * (1.0 / N)
    xc = jnp.minimum(jnp.maximum(x2_ref[:, 0, :], 1e-6), hi)

    xm, xp = _shift_cols(xc, m_wm, m_wp, HW)
    mids = []
    for o in range(Cout):
        s1 = g1_ref[o] * jax.lax.rsqrt(v1_ref[o] + _EPS)
        w9 = [w1_ref[o * 9 + k] * s1 for k in range(9)]
        b1e = (b1_ref[o] - m1_ref[o]) * s1 + be1_ref[o]
        mids.append(_conv3x3_s(xc, xm, xp, w9, b1e, masks, W, HW))

    cms, cps = [], []
    for c in range(Cout):
        cm, cp = _shift_cols(mids[c], m_wm, m_wp, HW)
        cms.append(cm)
        cps.append(cp)
    for o in range(Cout):
        s2 = g2_ref[o] * jax.lax.rsqrt(v2_ref[o] + _EPS)
        qs = [None] * 3
        for c in range(Cout):
            base = (o * Cout + c) * 9
            for i in range(3):
                q = ((w2_ref[base + i * 3] * s2) * cms[c]
                     + (w2_ref[base + i * 3 + 1] * s2) * mids[c]
                     + (w2_ref[base + i * 3 + 2] * s2) * cps[c])
                qs[i] = q if qs[i] is None else qs[i] + q
        b2e = (b2_ref[o] - m2_ref[o]) * s2 + be2_ref[o]
        y = _row_combine(qs[0], qs[1], qs[2], m_hm, m_hp, W, HW)
        out_ref[:, o, :] = jnp.maximum(y + b2e, 0.0)


def _pick_nb(n, cap=8):
    for d in range(min(cap, n), 0, -1):
        if n % d == 0:
            return d
    return n


def kernel(x, pre_w, pre_b, pre_bn_gamma, pre_bn_beta, pre_bn_mean, pre_bn_var,
           ca_w, ca_b, ca_bn_gamma, ca_bn_beta, ca_bn_mean, ca_bn_var,
           cb_w, cb_b, cb_bn_gamma, cb_bn_beta, cb_bn_mean, cb_bn_var,
           att_c1_w, att_c1_b, att_bn_gamma, att_bn_beta, att_bn_mean,
           att_bn_var, att_ch_w, att_ch_b, att_cw_w, att_cw_b,
           th1_w, th1_b, th2_w, th2_b,
           post1_w, post1_b, post1_bn_gamma, post1_bn_beta, post1_bn_mean,
           post1_bn_var, post2_w, post2_b, post2_bn_gamma, post2_bn_beta,
           post2_bn_mean, post2_bn_var):
    N, Cin, H, W = x.shape
    HW = H * W
    Cout = int(post1_w.shape[0])
    nb1 = _pick_nb(N, cap=16)
    nb2 = _pick_nb(N, cap=8)

    f32 = jnp.float32
    s1_kern = functools.partial(_stage1_body, H=H, W=W)
    x2_3d, t = pl.pallas_call(
        s1_kern,
        out_shape=(jax.ShapeDtypeStruct((N, H, W), f32),
                   jax.ShapeDtypeStruct((N, 1, 1), f32)),
        grid=(N // nb1,),
        in_specs=[
            pl.BlockSpec((nb1, Cin, H, W), lambda n: (n, 0, 0, 0)),
        ] + [_smem()] * 32,
        out_specs=(pl.BlockSpec((nb1, H, W), lambda n: (n, 0, 0)),
                   pl.BlockSpec((nb1, 1, 1), lambda n: (n, 0, 0))),
        compiler_params=_cparams(),
    )(x,
      pre_w.reshape(-1).astype(f32), pre_b.reshape(-1).astype(f32),
      pre_bn_gamma.astype(f32), pre_bn_beta.astype(f32),
      pre_bn_mean.astype(f32), pre_bn_var.astype(f32),
      ca_w.reshape(-1).astype(f32), ca_b.reshape(-1).astype(f32),
      ca_bn_gamma.astype(f32), ca_bn_beta.astype(f32),
      ca_bn_mean.astype(f32), ca_bn_var.astype(f32),
      cb_w.reshape(-1).astype(f32), cb_b.reshape(-1).astype(f32),
      cb_bn_gamma.astype(f32), cb_bn_beta.astype(f32),
      cb_bn_mean.astype(f32), cb_bn_var.astype(f32),
      att_c1_w.reshape(-1).astype(f32), att_c1_b.reshape(-1).astype(f32),
      att_bn_gamma.astype(f32), att_bn_beta.astype(f32),
      att_bn_mean.astype(f32), att_bn_var.astype(f32),
      att_ch_w.reshape(-1).astype(f32), att_ch_b.reshape(-1).astype(f32),
      att_cw_w.reshape(-1).astype(f32), att_cw_b.reshape(-1).astype(f32),
      th1_w.reshape(-1).astype(f32), th1_b.reshape(-1).astype(f32),
      th2_w.reshape(-1).astype(f32), th2_b.reshape(-1).astype(f32))

    x2 = x2_3d.reshape(N, 1, HW)
    s2_kern = functools.partial(_stage2_body, H=H, W=W, Cout=Cout)
    out = pl.pallas_call(
        s2_kern,
        out_shape=jax.ShapeDtypeStruct((N, Cout, HW), f32),
        grid=(N // nb2,),
        in_specs=[
            pl.BlockSpec((nb2, 1, HW), lambda n: (n, 0, 0)),
            pl.BlockSpec((N, 1, 1), lambda n: (0, 0, 0)),
        ] + [_smem()] * 12,
        out_specs=pl.BlockSpec((nb2, Cout, HW), lambda n: (n, 0, 0)),
        compiler_params=_cparams(),
    )(x2, t,
      post1_w.reshape(-1).astype(f32), post1_b.reshape(-1).astype(f32),
      post1_bn_gamma.astype(f32), post1_bn_beta.astype(f32),
      post1_bn_mean.astype(f32), post1_bn_var.astype(f32),
      post2_w.reshape(-1).astype(f32), post2_b.reshape(-1).astype(f32),
      post2_bn_gamma.astype(f32), post2_bn_beta.astype(f32),
      post2_bn_mean.astype(f32), post2_bn_var.astype(f32))

    return out.reshape(N, Cout, H, W)


# speedup vs baseline: 1.4636x; 1.1111x over previous
import functools

import jax
import jax.numpy as jnp
from jax.experimental import pallas as pl
from jax.experimental.pallas import tpu as pltpu

_EPS = 1e-5


def _smem():
    return pl.BlockSpec(memory_space=pltpu.MemorySpace.SMEM)


def _cparams():
    return pltpu.CompilerParams(dimension_semantics=("parallel",),
                                vmem_limit_bytes=48 * 1024 * 1024)


def _param_layout(Cin, Cout):
    sizes = (("pre_w", Cin), ("pre_b", 1), ("pre_g", 1), ("pre_be", 1),
             ("pre_m", 1), ("pre_v", 1),
             ("ca_w", 9), ("ca_b", 1), ("ca_g", 1), ("ca_be", 1),
             ("ca_m", 1), ("ca_v", 1),
             ("cb_w", 9), ("cb_b", 1), ("cb_g", 1), ("cb_be", 1),
             ("cb_m", 1), ("cb_v", 1),
             ("a1w", 8), ("a1b", 8), ("ag", 8), ("abe", 8), ("am", 8),
             ("av", 8),
             ("achw", 8), ("achb", 1), ("acww", 8), ("acwb", 1),
             ("t1w", 1), ("t1b", 1), ("t2w", 1), ("t2b", 1),
             ("w1", 9 * Cout), ("b1", Cout), ("g1", Cout), ("be1", Cout),
             ("m1", Cout), ("v1", Cout),
             ("w2", 9 * Cout * Cout), ("b2", Cout), ("g2", Cout),
             ("be2", Cout), ("m2", Cout), ("v2", Cout))
    off = {}
    k = 0
    for name, n in sizes:
        off[name] = k
        k += n
    return off


def _shift_w3(v, d):
    nb, H, W = v.shape
    z = jnp.zeros((nb, H, 1), jnp.float32)
    if d == -1:
        return jnp.concatenate([z, v[:, :, :W - 1]], axis=2)
    return jnp.concatenate([v[:, :, 1:], z], axis=2)


def _shift_h3(v, d):
    nb, H, W = v.shape
    z = jnp.zeros((nb, 1, W), jnp.float32)
    if d == -1:
        return jnp.concatenate([z, v[:, :H - 1, :]], axis=1)
    return jnp.concatenate([v[:, 1:, :], z], axis=1)


def _conv3x3_hw(x, xm, xp, w9, bias):
    qs = [w9[i * 3] * xm + w9[i * 3 + 1] * x + w9[i * 3 + 2] * xp
          for i in range(3)]
    y = qs[1] + _shift_h3(qs[0], -1) + _shift_h3(qs[2], 1)
    return jnp.maximum(y + bias, 0.0)


def _stage1_body(x_ref, p_ref, x2_ref, t_ref, *, H, W, Cout):
    nb, Cin = x_ref.shape[0], x_ref.shape[1]
    HW = H * W
    off = _param_layout(Cin, Cout)

    def p(name, i=0):
        return p_ref[off[name] + i]

    pscale = p("pre_g") * jax.lax.rsqrt(p("pre_v") + _EPS)
    acc = (p("pre_w", 0) * pscale) * x_ref[:, 0, :, :]
    for c in range(1, Cin):
        acc = acc + (p("pre_w", c) * pscale) * x_ref[:, c, :, :]
    pbias = (p("pre_b") - p("pre_m")) * pscale + p("pre_be")
    x1 = jnp.maximum(acc + pbias, 0.0)

    cas = p("ca_g") * jax.lax.rsqrt(p("ca_v") + _EPS)
    caw = [p("ca_w", k) * cas for k in range(9)]
    cabias = (p("ca_b") - p("ca_m")) * cas + p("ca_be")
    cbs = p("cb_g") * jax.lax.rsqrt(p("cb_v") + _EPS)
    cbw = [p("cb_w", k) * cbs for k in range(9)]
    cbbias = (p("cb_b") - p("cb_m")) * cbs + p("cb_be")

    y = _conv3x3_hw(x1, _shift_w3(x1, -1), _shift_w3(x1, 1), caw, cabias)
    y = _conv3x3_hw(y, _shift_w3(y, -1), _shift_w3(y, 1), cbw, cbbias)
    x2 = x1 + y
    x2_ref[...] = x2

    xh = jnp.sum(x2, axis=2) * (1.0 / W)
    xw = jnp.sum(x2, axis=1) * (1.0 / H)
    xcat = jnp.concatenate([xh, xw], axis=1)
    lane = jax.lax.broadcasted_iota(jnp.int32, (1, H + W), 1)
    is_h = lane < H
    acc_a = None
    for m in range(8):
        asc = p("ag", m) * jax.lax.rsqrt(p("av", m) + _EPS)
        wm = p("a1w", m) * asc
        bm = (p("a1b", m) - p("am", m)) * asc + p("abe", m)
        z = wm * xcat + bm
        z = z * jnp.clip(z + 3.0, 0.0, 6.0) * (1.0 / 6.0)
        contrib = jnp.where(is_h, p("achw", m), p("acww", m)) * z
        acc_a = contrib if acc_a is None else acc_a + contrib
    bsel = jnp.where(is_h, p("achb"), p("acwb"))
    a_cat = 1.0 / (1.0 + jnp.exp(-(acc_a + bsel)))
    a_h = a_cat[:, :H]
    a_w = a_cat[:, H:]

    row = jnp.sum(x2 * a_w[:, None, :], axis=2)
    sq = jnp.sum(row * a_h, axis=1, keepdims=True) * (1.0 / HW)

    tt = jnp.maximum(sq * p("t1w") + p("t1b"), 0.0)
    tt = 1.0 / (1.0 + jnp.exp(-(tt * p("t2w") + p("t2b"))))
    t_ref[...] = tt.reshape(t_ref.shape)


def _edge_masks(H, W, HW):
    lane = jax.lax.broadcasted_iota(jnp.int32, (1, HW), 1)
    col = jax.lax.rem(lane, W)
    row = lane // W
    return (col > 0), (col < W - 1), (row > 0), (row < H - 1)


def _shift_cols(x, m_wm, m_wp, HW):
    xm = jnp.where(m_wm, pltpu.roll(x, shift=1, axis=1), 0.0)
    xp = jnp.where(m_wp, pltpu.roll(x, shift=HW - 1, axis=1), 0.0)
    return xm, xp


def _row_combine(q_up, q_mid, q_dn, m_hm, m_hp, W, HW):
    out = q_mid + jnp.where(m_hm, pltpu.roll(q_up, shift=W, axis=1), 0.0)
    return out + jnp.where(m_hp, pltpu.roll(q_dn, shift=HW - W, axis=1), 0.0)


def _conv3x3_s(x, xm, xp, w9, bias, masks, W, HW):
    m_wm, m_wp, m_hm, m_hp = masks
    qs = [w9[i * 3] * xm + w9[i * 3 + 1] * x + w9[i * 3 + 2] * xp
          for i in range(3)]
    y = _row_combine(qs[0], qs[1], qs[2], m_hm, m_hp, W, HW)
    return jnp.maximum(y + bias, 0.0)


def _stage2_body(x2_ref, t_ref, p_ref, out_ref, *, H, W, Cin, Cout):
    nb, _, HW = x2_ref.shape
    N = t_ref.shape[0]
    off = _param_layout(Cin, Cout)

    def p(name, i=0):
        return p_ref[off[name] + i]

    masks = _edge_masks(H, W, HW)
    m_wm, m_wp, m_hm, m_hp = masks

    hi = jnp.sum(t_ref[...]) * (1.0 / N)
    xc = jnp.minimum(jnp.maximum(x2_ref[:, 0, :], 1e-6), hi)

    xm, xp = _shift_cols(xc, m_wm, m_wp, HW)
    mids = []
    for o in range(Cout):
        s1 = p("g1", o) * jax.lax.rsqrt(p("v1", o) + _EPS)
        w9 = [p("w1", o * 9 + k) * s1 for k in range(9)]
        b1e = (p("b1", o) - p("m1", o)) * s1 + p("be1", o)
        mids.append(_conv3x3_s(xc, xm, xp, w9, b1e, masks, W, HW))

    cms, cps = [], []
    for c in range(Cout):
        cm, cp = _shift_cols(mids[c], m_wm, m_wp, HW)
        cms.append(cm)
        cps.append(cp)
    for o in range(Cout):
        s2 = p("g2", o) * jax.lax.rsqrt(p("v2", o) + _EPS)
        qs = [None] * 3
        for c in range(Cout):
            base = (o * Cout + c) * 9
            for i in range(3):
                q = ((p("w2", base + i * 3) * s2) * cms[c]
                     + (p("w2", base + i * 3 + 1) * s2) * mids[c]
                     + (p("w2", base + i * 3 + 2) * s2) * cps[c])
                qs[i] = q if qs[i] is None else qs[i] + q
        b2e = (p("b2", o) - p("m2", o)) * s2 + p("be2", o)
        y = _row_combine(qs[0], qs[1], qs[2], m_hm, m_hp, W, HW)
        out_ref[:, o, :] = jnp.maximum(y + b2e, 0.0)


def _pick_nb(n, cap=8):
    for d in range(min(cap, n), 0, -1):
        if n % d == 0:
            return d
    return n


def kernel(x, pre_w, pre_b, pre_bn_gamma, pre_bn_beta, pre_bn_mean, pre_bn_var,
           ca_w, ca_b, ca_bn_gamma, ca_bn_beta, ca_bn_mean, ca_bn_var,
           cb_w, cb_b, cb_bn_gamma, cb_bn_beta, cb_bn_mean, cb_bn_var,
           att_c1_w, att_c1_b, att_bn_gamma, att_bn_beta, att_bn_mean,
           att_bn_var, att_ch_w, att_ch_b, att_cw_w, att_cw_b,
           th1_w, th1_b, th2_w, th2_b,
           post1_w, post1_b, post1_bn_gamma, post1_bn_beta, post1_bn_mean,
           post1_bn_var, post2_w, post2_b, post2_bn_gamma, post2_bn_beta,
           post2_bn_mean, post2_bn_var):
    N, Cin, H, W = x.shape
    HW = H * W
    Cout = int(post1_w.shape[0])
    nb1 = _pick_nb(N, cap=8)
    nb2 = _pick_nb(N, cap=8)

    f32 = jnp.float32
    parts = [pre_w, pre_b, pre_bn_gamma, pre_bn_beta, pre_bn_mean, pre_bn_var,
             ca_w, ca_b, ca_bn_gamma, ca_bn_beta, ca_bn_mean, ca_bn_var,
             cb_w, cb_b, cb_bn_gamma, cb_bn_beta, cb_bn_mean, cb_bn_var,
             att_c1_w, att_c1_b, att_bn_gamma, att_bn_beta, att_bn_mean,
             att_bn_var, att_ch_w, att_ch_b, att_cw_w, att_cw_b,
             th1_w, th1_b, th2_w, th2_b,
             post1_w, post1_b, post1_bn_gamma, post1_bn_beta, post1_bn_mean,
             post1_bn_var, post2_w, post2_b, post2_bn_gamma, post2_bn_beta,
             post2_bn_mean, post2_bn_var]
    pvec = jnp.concatenate([a.reshape(-1).astype(f32) for a in parts])

    s1_kern = functools.partial(_stage1_body, H=H, W=W, Cout=Cout)
    x2_3d, t = pl.pallas_call(
        s1_kern,
        out_shape=(jax.ShapeDtypeStruct((N, H, W), f32),
                   jax.ShapeDtypeStruct((N, 1, 1), f32)),
        grid=(N // nb1,),
        in_specs=[
            pl.BlockSpec((nb1, Cin, H, W), lambda n: (n, 0, 0, 0)),
            _smem(),
        ],
        out_specs=(pl.BlockSpec((nb1, H, W), lambda n: (n, 0, 0)),
                   pl.BlockSpec((nb1, 1, 1), lambda n: (n, 0, 0))),
        compiler_params=_cparams(),
    )(x, pvec)

    x2 = x2_3d.reshape(N, 1, HW)
    s2_kern = functools.partial(_stage2_body, H=H, W=W, Cin=Cin, Cout=Cout)
    out = pl.pallas_call(
        s2_kern,
        out_shape=jax.ShapeDtypeStruct((N, Cout, HW), f32),
        grid=(N // nb2,),
        in_specs=[
            pl.BlockSpec((nb2, 1, HW), lambda n: (n, 0, 0)),
            pl.BlockSpec((N, 1, 1), lambda n: (0, 0, 0)),
            _smem(),
        ],
        out_specs=pl.BlockSpec((nb2, Cout, HW), lambda n: (n, 0, 0)),
        compiler_params=_cparams(),
    )(x2, t, pvec)

    return out.reshape(N, Cout, H, W)
